# baseline (speedup 1.0000x reference)
import functools

import jax
import jax.numpy as jnp
from jax import lax
from jax.experimental import pallas as pl
from jax.experimental.pallas import tpu as pltpu
from jax.experimental.pallas import tpu_sc as plsc

_CHUNK = 48
_RPT = 192
_SCHUNK = 120
_N_ISSUERS = 2


def _chunks(total, chunk):
    sizes = []
    left = total
    while left > 0:
        sizes.append(min(chunk, left))
        left -= sizes[-1]
    return sizes


def _copy_fanout(w_hbm, out_hbm, bufs, in_sem, out_sem, base, sizes, batch,
                 seq_len):
    offs = [0]
    for s in sizes[:-1]:
        offs.append(offs[-1] + s)
    n = len(sizes)
    pltpu.async_copy(
        w_hbm.at[pl.ds(base + offs[0], sizes[0])],
        bufs[0].at[pl.ds(0, sizes[0])], in_sem,
    )
    for i in range(n):
        cur = bufs[i % 2].at[pl.ds(0, sizes[i])]
        r0 = base + offs[i]
        pltpu.make_async_copy(w_hbm.at[pl.ds(r0, sizes[i])], cur,
                              in_sem).wait()
        for b in range(batch):
            pltpu.async_copy(
                cur, out_hbm.at[pl.ds(b * seq_len + r0, sizes[i])], out_sem
            )
        if i + 1 < n:
            pltpu.async_copy(
                w_hbm.at[pl.ds(base + offs[i + 1], sizes[i + 1])],
                bufs[(i + 1) % 2].at[pl.ds(0, sizes[i + 1])],
                in_sem,
            )
        for b in range(batch):
            pltpu.make_async_copy(
                cur, out_hbm.at[pl.ds(b * seq_len + r0, sizes[i])], out_sem
            ).wait()


@functools.partial(jax.jit, static_argnums=(1, 2))
def _sc_broadcast_rows(W, batch, seq_len):
    d = W.shape[1]
    info = plsc.get_sparse_core_info()
    nc, ns = info.num_cores, info.num_subcores
    nw = nc * ns

    n_tile_workers = nw - nc * _N_ISSUERS
    rows_tile = n_tile_workers * _RPT
    rows_spmem = seq_len - rows_tile
    rows_issuer = rows_spmem // (nc * _N_ISSUERS)
    assert rows_spmem > 0 and rows_spmem % (nc * _N_ISSUERS) == 0
    tile_sizes = _chunks(_RPT, _CHUNK)
    spmem_sizes = _chunks(rows_issuer, _SCHUNK)

    mesh = plsc.VectorSubcoreMesh(core_axis_name="c", subcore_axis_name="s")

    @functools.partial(
        pl.kernel,
        mesh=mesh,
        out_type=jax.ShapeDtypeStruct((batch * seq_len, d), jnp.float32),
        scratch_types=[
            pltpu.VMEM((_CHUNK, d), jnp.float32),
            pltpu.VMEM((_CHUNK, d), jnp.float32),
            pltpu.VMEM_SHARED((_N_ISSUERS, 2, _SCHUNK, d), jnp.float32),
            pltpu.SemaphoreType.DMA,
            pltpu.SemaphoreType.DMA,
            pltpu.SemaphoreType.DMA,
            pltpu.SemaphoreType.DMA,
        ],
    )
    def k(w_hbm, out_hbm, buf0, buf1, sbuf, in_sem, out_sem, s_in_sem,
          s_out_sem):
        cid = lax.axis_index("c")
        sid = lax.axis_index("s")

        @pl.when(sid < _N_ISSUERS)
        def _spmem_path():
            issuer = cid * _N_ISSUERS + sid
            sbase = issuer * rows_issuer
            sbufs = (sbuf.at[sid, 0], sbuf.at[sid, 1])
            _copy_fanout(w_hbm, out_hbm, sbufs, s_in_sem, s_out_sem, sbase,
                         spmem_sizes, batch, seq_len)

        @pl.when(sid >= _N_ISSUERS)
        def _tile_path():
            tid = (sid - _N_ISSUERS) * nc + cid
            base = rows_spmem + tid * _RPT
            _copy_fanout(w_hbm, out_hbm, (buf0, buf1), in_sem, out_sem, base,
                         tile_sizes, batch, seq_len)

    return k(W)


def kernel(x, W):
    batch, seq_len = x.shape
    d = W.shape[1]
    flat = _sc_broadcast_rows(W, batch, seq_len)
    return flat.reshape(batch, seq_len, d)


# speedup vs baseline: 1.0181x; 1.0181x over previous
import functools

import jax
import jax.numpy as jnp
from jax import lax
from jax.experimental import pallas as pl
from jax.experimental.pallas import tpu as pltpu
from jax.experimental.pallas import tpu_sc as plsc

_CHUNK = 64


@functools.partial(jax.jit, static_argnums=(1, 2))
def _sc_broadcast_rows(W, batch, seq_len):
    d = W.shape[1]
    info = plsc.get_sparse_core_info()
    nw = info.num_cores * info.num_subcores
    rows_per_w = seq_len // nw
    chunk = min(_CHUNK, rows_per_w)
    n_chunks = rows_per_w // chunk
    mesh = plsc.VectorSubcoreMesh(core_axis_name="c", subcore_axis_name="s")

    @functools.partial(
        pl.kernel,
        mesh=mesh,
        out_type=jax.ShapeDtypeStruct((batch * seq_len, d), jnp.float32),
        scratch_types=[
            pltpu.VMEM((chunk, d), jnp.float32),
            pltpu.VMEM((chunk, d), jnp.float32),
            pltpu.SemaphoreType.DMA,
            pltpu.SemaphoreType.DMA,
        ],
    )
    def k(w_hbm, out_hbm, buf0, buf1, in_sem, out_sem):
        wid = lax.axis_index("s") * info.num_cores + lax.axis_index("c")
        base = wid * rows_per_w
        bufs = (buf0, buf1)

        pltpu.async_copy(w_hbm.at[pl.ds(base, chunk)], buf0, in_sem)

        for i in range(n_chunks):
            cur = bufs[i % 2]
            pltpu.make_async_copy(
                w_hbm.at[pl.ds(base + i * chunk, chunk)], cur, in_sem
            ).wait()
            r0 = base + i * chunk
            for b in range(batch):
                pltpu.async_copy(
                    cur, out_hbm.at[pl.ds(b * seq_len + r0, chunk)], out_sem
                )
            if i + 1 < n_chunks:
                pltpu.async_copy(
                    w_hbm.at[pl.ds(base + (i + 1) * chunk, chunk)],
                    bufs[(i + 1) % 2],
                    in_sem,
                )
            for b in range(batch):
                pltpu.make_async_copy(
                    cur, out_hbm.at[pl.ds(b * seq_len + r0, chunk)], out_sem
                ).wait()

    return k(W)


def kernel(x, W):
    batch, seq_len = x.shape
    d = W.shape[1]
    flat = _sc_broadcast_rows(W, batch, seq_len)
    return flat.reshape(batch, seq_len, d)
